# Initial kernel scaffold; baseline (speedup 1.0000x reference)
import jax
import jax.numpy as jnp
from jax.experimental import pallas as pl


def kernel(x, Wp_w, Wp_b, C, Th_w, Th_b):
    raise NotImplementedError("write your pallas kernel here")


# speedup vs baseline: 7.6127x; 7.6127x over previous
import jax
import jax.numpy as jnp
from jax.experimental import pallas as pl
from jax.experimental.pallas import tpu as pltpu

N = 1024
D = 128
P = 64
NC = 10
K = 10
OUT = 128

_HIGH = jax.lax.Precision.HIGHEST


def _dot(a, b, dims):
    return jax.lax.dot_general(a, b, (dims, ((), ())),
                               preferred_element_type=jnp.float32,
                               precision=_HIGH)


def _bdot(a, b, dims):
    return jax.lax.dot_general(a.astype(jnp.bfloat16), b.astype(jnp.bfloat16),
                               (dims, ((), ())),
                               preferred_element_type=jnp.float32)


def _tc_body(x_ref, wpw_ref, wpb_ref, c_ref, thw_ref, thb_ref, out_ref,
             d2_ref, h_ref):
    xb = x_ref[0]
    Z = _bdot(xb, wpw_ref[...], ((1,), (1,))) + wpb_ref[...]

    ones_col = jnp.ones((N, 1), dtype=jnp.float32)
    sq = jnp.sum(Z * Z, axis=1, keepdims=True)
    G = _bdot(Z, Z, ((1,), (1,)))
    sq_row = _dot(ones_col, sq, ((1,), (1,)))
    d2_ref[...] = (sq - 2.0 * G) + sq_row

    iota_cols = jax.lax.broadcasted_iota(jnp.int32, (N, N), 1)
    h_ref[...] = jnp.zeros((N, N), dtype=jnp.float32)
    for _ in range(K):
        d2v = d2_ref[...]
        m = jnp.min(d2v, axis=1, keepdims=True)
        cand = jnp.where(d2v <= m, iota_cols, N)
        am = jnp.min(cand, axis=1, keepdims=True)
        onehot = iota_cols == am
        h_ref[...] = h_ref[...] + onehot.astype(jnp.float32)
        d2_ref[...] = jnp.where(onehot, jnp.float32(jnp.inf), d2v)

    S = _bdot(Z, c_ref[...], ((1,), (1,)))
    mx = jnp.max(S, axis=1, keepdims=True)
    e = jnp.exp(S - mx)
    Hc = e / jnp.sum(e, axis=1, keepdims=True)

    dv = jnp.float32(K) + jnp.sum(Hc, axis=1, keepdims=True)
    dvis = jax.lax.rsqrt(dv)

    Xt = _bdot(xb, thw_ref[...], ((1,), (1,))) + thb_ref[...]
    Xs = dvis * Xt

    h = h_ref[...]
    E_knn = _bdot(h, Xs, ((0,), (0,)))
    de_knn = _dot(h, ones_col, ((0,), (0,)))
    E_knn = E_knn * jnp.where(de_knn > 0, 1.0 / jnp.where(de_knn > 0, de_knn, 1.0), 0.0)

    E_c = _bdot(Hc, Xs, ((0,), (0,)))
    de_c = _dot(Hc, ones_col, ((0,), (0,)))
    E_c = E_c * jnp.where(de_c > 0, 1.0 / jnp.where(de_c > 0, de_c, 1.0), 0.0)

    o = _bdot(h, E_knn, ((1,), (0,))) + _bdot(Hc, E_c, ((1,), (0,)))
    o = dvis * o
    out_ref[0] = jnp.where(o > 0, o, jnp.exp(jnp.minimum(o, 0.0)) - 1.0)


def kernel(x, Wp_w, Wp_b, C, Th_w, Th_b):
    Bb, Nn, Mm, Dd = x.shape
    BM = Bb * Mm
    x_flat = jnp.transpose(x, (0, 2, 1, 3)).reshape(BM, Nn, Dd)
    wpb = Wp_b.reshape(1, P)
    thb = Th_b.reshape(1, OUT)

    bcast = lambda *shape: pl.BlockSpec(shape, lambda i: (0,) * len(shape))
    y = pl.pallas_call(
        _tc_body,
        grid=(BM,),
        in_specs=[
            pl.BlockSpec((1, N, D), lambda i: (i, 0, 0)),
            bcast(P, D),
            bcast(1, P),
            bcast(NC, P),
            bcast(OUT, D),
            bcast(1, OUT),
        ],
        out_specs=pl.BlockSpec((1, N, OUT), lambda i: (i, 0, 0)),
        out_shape=jax.ShapeDtypeStruct((BM, N, OUT), jnp.float32),
        scratch_shapes=[
            pltpu.VMEM((N, N), jnp.float32),
            pltpu.VMEM((N, N), jnp.float32),
        ],
        compiler_params=pltpu.CompilerParams(
            dimension_semantics=("arbitrary",),
        ),
    )(x_flat, Wp_w, wpb, C, Th_w, thb)

    return jnp.transpose(y.reshape(Bb, Mm, Nn, OUT), (0, 2, 1, 3))


# speedup vs baseline: 9.7031x; 1.2746x over previous
import jax
import jax.numpy as jnp
from jax.experimental import pallas as pl
from jax.experimental.pallas import tpu as pltpu

N = 1024
D = 128
P = 64
NC = 10
K = 10
OUT = 128

_HIGH = jax.lax.Precision.HIGHEST


def _dot(a, b, dims):
    return jax.lax.dot_general(a, b, (dims, ((), ())),
                               preferred_element_type=jnp.float32,
                               precision=_HIGH)


def _bdot(a, b, dims):
    return jax.lax.dot_general(a.astype(jnp.bfloat16), b.astype(jnp.bfloat16),
                               (dims, ((), ())),
                               preferred_element_type=jnp.float32)


def _tc_body(x_ref, wpw_ref, wpb_ref, c_ref, thw_ref, thb_ref, out_ref,
             d2_ref, h_ref):
    xb = x_ref[0]
    Z = _bdot(xb, wpw_ref[...], ((1,), (1,))) + wpb_ref[...]

    ones_col = jnp.ones((N, 1), dtype=jnp.float32)
    sq = jnp.sum(Z * Z, axis=1, keepdims=True)
    G = _bdot(Z, Z, ((1,), (1,)))
    sq_row = _dot(ones_col, sq, ((1,), (1,)))
    d2_ref[...] = (sq - 2.0 * G) + sq_row

    d2v = d2_ref[...]
    m = jnp.min(d2v, axis=1, keepdims=True)
    h_ref[...] = (d2v == m).astype(jnp.float32)
    for _ in range(K - 1):
        t = d2_ref[...] + h_ref[...] * jnp.float32(1e30)
        m = jnp.min(t, axis=1, keepdims=True)
        h_ref[...] = h_ref[...] + (t == m).astype(jnp.float32)

    S = _bdot(Z, c_ref[...], ((1,), (1,)))
    mx = jnp.max(S, axis=1, keepdims=True)
    e = jnp.exp(S - mx)
    Hc = e / jnp.sum(e, axis=1, keepdims=True)

    dv = jnp.float32(K) + jnp.sum(Hc, axis=1, keepdims=True)
    dvis = jax.lax.rsqrt(dv)

    Xt = _bdot(xb, thw_ref[...], ((1,), (1,))) + thb_ref[...]
    Xs = dvis * Xt

    h = h_ref[...]
    E_knn = _bdot(h, Xs, ((0,), (0,)))
    de_knn = _dot(h, ones_col, ((0,), (0,)))
    E_knn = E_knn * jnp.where(de_knn > 0, 1.0 / jnp.where(de_knn > 0, de_knn, 1.0), 0.0)

    E_c = _bdot(Hc, Xs, ((0,), (0,)))
    de_c = _dot(Hc, ones_col, ((0,), (0,)))
    E_c = E_c * jnp.where(de_c > 0, 1.0 / jnp.where(de_c > 0, de_c, 1.0), 0.0)

    o = _bdot(h, E_knn, ((1,), (0,))) + _bdot(Hc, E_c, ((1,), (0,)))
    o = dvis * o
    out_ref[0] = jnp.where(o > 0, o, jnp.exp(jnp.minimum(o, 0.0)) - 1.0)


def kernel(x, Wp_w, Wp_b, C, Th_w, Th_b):
    Bb, Nn, Mm, Dd = x.shape
    BM = Bb * Mm
    x_flat = jnp.transpose(x, (0, 2, 1, 3)).reshape(BM, Nn, Dd)
    wpb = Wp_b.reshape(1, P)
    thb = Th_b.reshape(1, OUT)

    bcast = lambda *shape: pl.BlockSpec(shape, lambda i: (0,) * len(shape))
    y = pl.pallas_call(
        _tc_body,
        grid=(BM,),
        in_specs=[
            pl.BlockSpec((1, N, D), lambda i: (i, 0, 0)),
            bcast(P, D),
            bcast(1, P),
            bcast(NC, P),
            bcast(OUT, D),
            bcast(1, OUT),
        ],
        out_specs=pl.BlockSpec((1, N, OUT), lambda i: (i, 0, 0)),
        out_shape=jax.ShapeDtypeStruct((BM, N, OUT), jnp.float32),
        scratch_shapes=[
            pltpu.VMEM((N, N), jnp.float32),
            pltpu.VMEM((N, N), jnp.float32),
        ],
        compiler_params=pltpu.CompilerParams(
            dimension_semantics=("arbitrary",),
        ),
    )(x_flat, Wp_w, wpb, C, Th_w, thb)

    return jnp.transpose(y.reshape(Bb, Mm, Nn, OUT), (0, 2, 1, 3))


# speedup vs baseline: 16.3572x; 1.6858x over previous
import jax
import jax.numpy as jnp
from jax.experimental import pallas as pl
from jax.experimental.pallas import tpu as pltpu

N = 1024
D = 128
P = 64
NC = 10
K = 10
OUT = 128

_HIGH = jax.lax.Precision.HIGHEST


def _dot(a, b, dims):
    return jax.lax.dot_general(a, b, (dims, ((), ())),
                               preferred_element_type=jnp.float32,
                               precision=_HIGH)


def _bdot(a, b, dims):
    return jax.lax.dot_general(a.astype(jnp.bfloat16), b.astype(jnp.bfloat16),
                               (dims, ((), ())),
                               preferred_element_type=jnp.float32)


def _tc_body(x_ref, wpw_ref, wpb_ref, c_ref, thw_ref, thb_ref, out_ref,
             d2_ref, h_ref):
    xb = x_ref[0]
    Z = _bdot(xb, wpw_ref[...], ((1,), (1,))) + wpb_ref[...]

    ones_col = jnp.ones((N, 1), dtype=jnp.float32)
    sq = jnp.sum(Z * Z, axis=1, keepdims=True)
    G = _bdot(Z, Z, ((1,), (1,)))
    sq_row = _dot(ones_col, sq, ((1,), (1,)))
    d2_ref[...] = (sq - 2.0 * G) + sq_row

    d2v = d2_ref[...]
    m = jnp.min(d2v, axis=1, keepdims=True)
    for _ in range(K - 1):
        t = jnp.where(d2v <= m, jnp.float32(jnp.inf), d2v)
        m = jnp.min(t, axis=1, keepdims=True)
    h_ref[...] = (d2v <= m).astype(jnp.bfloat16)

    S = _bdot(Z, c_ref[...], ((1,), (1,)))
    mx = jnp.max(S, axis=1, keepdims=True)
    e = jnp.exp(S - mx)
    Hc = e / jnp.sum(e, axis=1, keepdims=True)

    dv = jnp.float32(K) + jnp.sum(Hc, axis=1, keepdims=True)
    dvis = jax.lax.rsqrt(dv)

    Xt = _bdot(xb, thw_ref[...], ((1,), (1,))) + thb_ref[...]
    Xs = dvis * Xt

    h = h_ref[...]
    E_knn = _bdot(h, Xs, ((0,), (0,)))
    de_knn = _bdot(h, ones_col, ((0,), (0,)))
    E_knn = E_knn * jnp.where(de_knn > 0, 1.0 / jnp.where(de_knn > 0, de_knn, 1.0), 0.0)

    E_c = _bdot(Hc, Xs, ((0,), (0,)))
    de_c = _dot(Hc, ones_col, ((0,), (0,)))
    E_c = E_c * jnp.where(de_c > 0, 1.0 / jnp.where(de_c > 0, de_c, 1.0), 0.0)

    o = _bdot(h, E_knn, ((1,), (0,))) + _bdot(Hc, E_c, ((1,), (0,)))
    o = dvis * o
    out_ref[0] = jnp.where(o > 0, o, jnp.exp(jnp.minimum(o, 0.0)) - 1.0)


def kernel(x, Wp_w, Wp_b, C, Th_w, Th_b):
    Bb, Nn, Mm, Dd = x.shape
    BM = Bb * Mm
    x_flat = jnp.transpose(x, (0, 2, 1, 3)).reshape(BM, Nn, Dd)
    wpb = Wp_b.reshape(1, P)
    thb = Th_b.reshape(1, OUT)

    bcast = lambda *shape: pl.BlockSpec(shape, lambda i: (0,) * len(shape))
    y = pl.pallas_call(
        _tc_body,
        grid=(BM,),
        in_specs=[
            pl.BlockSpec((1, N, D), lambda i: (i, 0, 0)),
            bcast(P, D),
            bcast(1, P),
            bcast(NC, P),
            bcast(OUT, D),
            bcast(1, OUT),
        ],
        out_specs=pl.BlockSpec((1, N, OUT), lambda i: (i, 0, 0)),
        out_shape=jax.ShapeDtypeStruct((BM, N, OUT), jnp.float32),
        scratch_shapes=[
            pltpu.VMEM((N, N), jnp.float32),
            pltpu.VMEM((N, N), jnp.bfloat16),
        ],
        compiler_params=pltpu.CompilerParams(
            dimension_semantics=("arbitrary",),
        ),
    )(x_flat, Wp_w, wpb, C, Th_w, thb)

    return jnp.transpose(y.reshape(Bb, Mm, Nn, OUT), (0, 2, 1, 3))


# speedup vs baseline: 16.5395x; 1.0111x over previous
import jax
import jax.numpy as jnp
from jax.experimental import pallas as pl
from jax.experimental.pallas import tpu as pltpu

N = 1024
D = 128
P = 64
NC = 10
K = 10
OUT = 128

_HIGH = jax.lax.Precision.HIGHEST


def _dot(a, b, dims):
    return jax.lax.dot_general(a, b, (dims, ((), ())),
                               preferred_element_type=jnp.float32,
                               precision=_HIGH)


def _bdot(a, b, dims):
    return jax.lax.dot_general(a.astype(jnp.bfloat16), b.astype(jnp.bfloat16),
                               (dims, ((), ())),
                               preferred_element_type=jnp.float32)


def _tc_body(x_ref, wpw_ref, wpb_ref, c_ref, thw_ref, thb_ref, out_ref,
             d2_ref, h_ref):
    xb = x_ref[0]
    Z = _bdot(xb, wpw_ref[...], ((1,), (1,))) + wpb_ref[...]

    ones_col = jnp.ones((N, 1), dtype=jnp.float32)
    sq = jnp.sum(Z * Z, axis=1, keepdims=True)
    G = _bdot(Z, Z, ((1,), (1,)))
    sq_row = _dot(ones_col, sq, ((1,), (1,)))
    d2_ref[...] = (sq - 2.0 * G) + sq_row

    d2v = d2_ref[...]
    m = jnp.min(d2v, axis=0, keepdims=True)
    for _ in range(K - 1):
        t = jnp.where(d2v <= m, jnp.float32(jnp.inf), d2v)
        m = jnp.min(t, axis=0, keepdims=True)
    h_ref[...] = (d2v <= m).astype(jnp.bfloat16)

    S = _bdot(Z, c_ref[...], ((1,), (1,)))
    mx = jnp.max(S, axis=1, keepdims=True)
    e = jnp.exp(S - mx)
    Hc = e / jnp.sum(e, axis=1, keepdims=True)

    dv = jnp.float32(K) + jnp.sum(Hc, axis=1, keepdims=True)
    dvis = jax.lax.rsqrt(dv)

    Xt = _bdot(xb, thw_ref[...], ((1,), (1,))) + thb_ref[...]
    Xs = dvis * Xt

    ht = h_ref[...]
    E_knn = _bdot(ht, Xs, ((1,), (0,)))
    de_knn = _bdot(ht, ones_col, ((1,), (0,)))
    E_knn = E_knn * jnp.where(de_knn > 0, 1.0 / jnp.where(de_knn > 0, de_knn, 1.0), 0.0)

    E_c = _bdot(Hc, Xs, ((0,), (0,)))
    de_c = _dot(Hc, ones_col, ((0,), (0,)))
    E_c = E_c * jnp.where(de_c > 0, 1.0 / jnp.where(de_c > 0, de_c, 1.0), 0.0)

    o = _bdot(ht, E_knn, ((0,), (0,))) + _bdot(Hc, E_c, ((1,), (0,)))
    o = dvis * o
    out_ref[0] = jnp.where(o > 0, o, jnp.exp(jnp.minimum(o, 0.0)) - 1.0)


def kernel(x, Wp_w, Wp_b, C, Th_w, Th_b):
    Bb, Nn, Mm, Dd = x.shape
    BM = Bb * Mm
    x_flat = jnp.transpose(x, (0, 2, 1, 3)).reshape(BM, Nn, Dd)
    wpb = Wp_b.reshape(1, P)
    thb = Th_b.reshape(1, OUT)

    bcast = lambda *shape: pl.BlockSpec(shape, lambda i: (0,) * len(shape))
    y = pl.pallas_call(
        _tc_body,
        grid=(BM,),
        in_specs=[
            pl.BlockSpec((1, N, D), lambda i: (i, 0, 0)),
            bcast(P, D),
            bcast(1, P),
            bcast(NC, P),
            bcast(OUT, D),
            bcast(1, OUT),
        ],
        out_specs=pl.BlockSpec((1, N, OUT), lambda i: (i, 0, 0)),
        out_shape=jax.ShapeDtypeStruct((BM, N, OUT), jnp.float32),
        scratch_shapes=[
            pltpu.VMEM((N, N), jnp.float32),
            pltpu.VMEM((N, N), jnp.bfloat16),
        ],
        compiler_params=pltpu.CompilerParams(
            dimension_semantics=("arbitrary",),
        ),
    )(x_flat, Wp_w, wpb, C, Th_w, thb)

    return jnp.transpose(y.reshape(Bb, Mm, Nn, OUT), (0, 2, 1, 3))


# speedup vs baseline: 21.9255x; 1.3256x over previous
import jax
import jax.numpy as jnp
from jax.experimental import pallas as pl
from jax.experimental.pallas import tpu as pltpu

N = 1024
D = 128
P = 64
NC = 10
K = 10
OUT = 128

_HIGH = jax.lax.Precision.HIGHEST


def _dot(a, b, dims):
    return jax.lax.dot_general(a, b, (dims, ((), ())),
                               preferred_element_type=jnp.float32,
                               precision=_HIGH)


def _bdot(a, b, dims):
    return jax.lax.dot_general(a.astype(jnp.bfloat16), b.astype(jnp.bfloat16),
                               (dims, ((), ())),
                               preferred_element_type=jnp.float32)


def _tc_body(x_ref, wpw_ref, wpb_ref, c_ref, thw_ref, thb_ref, out_ref,
             d2_ref, h_ref):
    xb = x_ref[0]
    Z = _bdot(xb, wpw_ref[...], ((1,), (1,))) + wpb_ref[...]

    ones_col = jnp.ones((N, 1), dtype=jnp.float32)
    sq = jnp.sum(Z * Z, axis=1, keepdims=True)
    G = _bdot(Z, Z, ((1,), (1,)))
    sq_row = jnp.transpose(sq)
    d2_ref[...] = (sq - 2.0 * G) + sq_row

    def _second_smallest(t):
        lo = jnp.minimum(t[:, : N // 2], t[:, N // 2:])
        hi = jnp.maximum(t[:, : N // 2], t[:, N // 2:])
        w = N // 4
        while w >= 128:
            a1, b1 = lo[:, :w], lo[:, w:]
            a2, b2 = hi[:, :w], hi[:, w:]
            lo = jnp.minimum(a1, b1)
            hi = jnp.minimum(jnp.maximum(a1, b1), jnp.minimum(a2, b2))
            w //= 2
        m1 = jnp.min(lo, axis=1, keepdims=True)
        t2 = jnp.where(lo == m1, hi, lo)
        return jnp.min(t2, axis=1, keepdims=True)

    d2v = d2_ref[...]
    m = _second_smallest(d2v)
    for _ in range(K // 2 - 1):
        t = jnp.where(d2v <= m, jnp.float32(jnp.inf), d2v)
        m = _second_smallest(t)
    h_ref[...] = (d2v <= m).astype(jnp.bfloat16)

    S = _bdot(Z, c_ref[...], ((1,), (1,)))
    mx = jnp.max(S, axis=1, keepdims=True)
    e = jnp.exp(S - mx)
    Hc = e / jnp.sum(e, axis=1, keepdims=True)

    dv = jnp.float32(K) + jnp.sum(Hc, axis=1, keepdims=True)
    dvis = jax.lax.rsqrt(dv)

    Xt = _bdot(xb, thw_ref[...], ((1,), (1,))) + thb_ref[...]
    Xs = dvis * Xt

    h = h_ref[...]
    E_knn = _bdot(h, Xs, ((0,), (0,)))
    de_knn = _bdot(h, ones_col, ((0,), (0,)))
    E_knn = E_knn * jnp.where(de_knn > 0, 1.0 / jnp.where(de_knn > 0, de_knn, 1.0), 0.0)

    E_c = _bdot(Hc, Xs, ((0,), (0,)))
    de_c = _dot(Hc, ones_col, ((0,), (0,)))
    E_c = E_c * jnp.where(de_c > 0, 1.0 / jnp.where(de_c > 0, de_c, 1.0), 0.0)

    o = _bdot(h, E_knn, ((1,), (0,))) + _bdot(Hc, E_c, ((1,), (0,)))
    o = dvis * o
    out_ref[0] = jnp.where(o > 0, o, jnp.exp(jnp.minimum(o, 0.0)) - 1.0)


def kernel(x, Wp_w, Wp_b, C, Th_w, Th_b):
    Bb, Nn, Mm, Dd = x.shape
    BM = Bb * Mm
    x_flat = jnp.transpose(x, (0, 2, 1, 3)).reshape(BM, Nn, Dd)
    wpb = Wp_b.reshape(1, P)
    thb = Th_b.reshape(1, OUT)

    bcast = lambda *shape: pl.BlockSpec(shape, lambda i: (0,) * len(shape))
    y = pl.pallas_call(
        _tc_body,
        grid=(BM,),
        in_specs=[
            pl.BlockSpec((1, N, D), lambda i: (i, 0, 0)),
            bcast(P, D),
            bcast(1, P),
            bcast(NC, P),
            bcast(OUT, D),
            bcast(1, OUT),
        ],
        out_specs=pl.BlockSpec((1, N, OUT), lambda i: (i, 0, 0)),
        out_shape=jax.ShapeDtypeStruct((BM, N, OUT), jnp.float32),
        scratch_shapes=[
            pltpu.VMEM((N, N), jnp.float32),
            pltpu.VMEM((N, N), jnp.bfloat16),
        ],
        compiler_params=pltpu.CompilerParams(
            dimension_semantics=("arbitrary",),
        ),
    )(x_flat, Wp_w, wpb, C, Th_w, thb)

    return jnp.transpose(y.reshape(Bb, Mm, Nn, OUT), (0, 2, 1, 3))


# speedup vs baseline: 22.4259x; 1.0228x over previous
import jax
import jax.numpy as jnp
from jax.experimental import pallas as pl
from jax.experimental.pallas import tpu as pltpu

N = 1024
D = 128
P = 64
NC = 10
K = 10
OUT = 128

_HIGH = jax.lax.Precision.HIGHEST


def _dot(a, b, dims):
    return jax.lax.dot_general(a, b, (dims, ((), ())),
                               preferred_element_type=jnp.float32,
                               precision=_HIGH)


def _bdot(a, b, dims):
    return jax.lax.dot_general(a.astype(jnp.bfloat16), b.astype(jnp.bfloat16),
                               (dims, ((), ())),
                               preferred_element_type=jnp.float32)


SLICES = 2


def _tc_body(x_ref, wpw_ref, wpb_ref, c_ref, thw_ref, thb_ref, out_ref,
             d2_ref, h_ref):
    for s in range(SLICES):
        _slice_pipe(s, x_ref, wpw_ref, wpb_ref, c_ref, thw_ref, thb_ref,
                    out_ref, d2_ref, h_ref)


def _slice_pipe(s, x_ref, wpw_ref, wpb_ref, c_ref, thw_ref, thb_ref, out_ref,
                d2_ref, h_ref):
    xb = x_ref[s]
    Z = _bdot(xb, wpw_ref[...], ((1,), (1,))) + wpb_ref[...]

    ones_col = jnp.ones((N, 1), dtype=jnp.float32)
    sq = jnp.sum(Z * Z, axis=1, keepdims=True)
    G = _bdot(Z, Z, ((1,), (1,)))
    sq_row = jnp.transpose(sq)
    d2_ref[s] = (sq - 2.0 * G) + sq_row

    def _second_smallest(t):
        lo = jnp.minimum(t[:, : N // 2], t[:, N // 2:])
        hi = jnp.maximum(t[:, : N // 2], t[:, N // 2:])
        w = N // 4
        while w >= 128:
            a1, b1 = lo[:, :w], lo[:, w:]
            a2, b2 = hi[:, :w], hi[:, w:]
            lo = jnp.minimum(a1, b1)
            hi = jnp.minimum(jnp.maximum(a1, b1), jnp.minimum(a2, b2))
            w //= 2
        m1 = jnp.min(lo, axis=1, keepdims=True)
        t2 = jnp.where(lo == m1, hi, lo)
        return jnp.min(t2, axis=1, keepdims=True)

    d2v = d2_ref[s]
    m = _second_smallest(d2v)
    for _ in range(K // 2 - 1):
        t = jnp.where(d2v <= m, jnp.float32(jnp.inf), d2v)
        m = _second_smallest(t)
    h_ref[s] = (d2v <= m).astype(jnp.bfloat16)

    S = _bdot(Z, c_ref[...], ((1,), (1,)))
    mx = jnp.max(S, axis=1, keepdims=True)
    e = jnp.exp(S - mx)
    Hc = e / jnp.sum(e, axis=1, keepdims=True)

    dv = jnp.float32(K) + jnp.sum(Hc, axis=1, keepdims=True)
    dvis = jax.lax.rsqrt(dv)

    Xt = _bdot(xb, thw_ref[...], ((1,), (1,))) + thb_ref[...]
    Xs = dvis * Xt

    h = h_ref[s]
    E_knn = _bdot(h, Xs, ((0,), (0,)))
    de_knn = _bdot(h, ones_col, ((0,), (0,)))
    E_knn = E_knn * jnp.where(de_knn > 0, 1.0 / jnp.where(de_knn > 0, de_knn, 1.0), 0.0)

    E_c = _bdot(Hc, Xs, ((0,), (0,)))
    de_c = _dot(Hc, ones_col, ((0,), (0,)))
    E_c = E_c * jnp.where(de_c > 0, 1.0 / jnp.where(de_c > 0, de_c, 1.0), 0.0)

    o = _bdot(h, E_knn, ((1,), (0,))) + _bdot(Hc, E_c, ((1,), (0,)))
    o = dvis * o
    out_ref[s] = jnp.where(o > 0, o, jnp.exp(jnp.minimum(o, 0.0)) - 1.0)


def kernel(x, Wp_w, Wp_b, C, Th_w, Th_b):
    Bb, Nn, Mm, Dd = x.shape
    BM = Bb * Mm
    x_flat = jnp.transpose(x, (0, 2, 1, 3)).reshape(BM, Nn, Dd)
    wpb = Wp_b.reshape(1, P)
    thb = Th_b.reshape(1, OUT)

    bcast = lambda *shape: pl.BlockSpec(shape, lambda i: (0,) * len(shape))
    y = pl.pallas_call(
        _tc_body,
        grid=(BM // SLICES,),
        in_specs=[
            pl.BlockSpec((SLICES, N, D), lambda i: (i, 0, 0)),
            bcast(P, D),
            bcast(1, P),
            bcast(NC, P),
            bcast(OUT, D),
            bcast(1, OUT),
        ],
        out_specs=pl.BlockSpec((SLICES, N, OUT), lambda i: (i, 0, 0)),
        out_shape=jax.ShapeDtypeStruct((BM, N, OUT), jnp.float32),
        scratch_shapes=[
            pltpu.VMEM((SLICES, N, N), jnp.float32),
            pltpu.VMEM((SLICES, N, N), jnp.bfloat16),
        ],
        compiler_params=pltpu.CompilerParams(
            dimension_semantics=("arbitrary",),
        ),
    )(x_flat, Wp_w, wpb, C, Th_w, thb)

    return jnp.transpose(y.reshape(Bb, Mm, Nn, OUT), (0, 2, 1, 3))


# speedup vs baseline: 23.2266x; 1.0357x over previous
import jax
import jax.numpy as jnp
from jax.experimental import pallas as pl
from jax.experimental.pallas import tpu as pltpu

N = 1024
D = 128
P = 64
NC = 10
K = 10
OUT = 128

_HIGH = jax.lax.Precision.HIGHEST


def _dot(a, b, dims):
    return jax.lax.dot_general(a, b, (dims, ((), ())),
                               preferred_element_type=jnp.float32,
                               precision=_HIGH)


def _bdot(a, b, dims):
    return jax.lax.dot_general(a.astype(jnp.bfloat16), b.astype(jnp.bfloat16),
                               (dims, ((), ())),
                               preferred_element_type=jnp.float32)


SLICES = 2


def _tc_body(x_ref, wpw_ref, wpb_ref, c_ref, thw_ref, thb_ref, out_ref,
             d2_ref, h_ref):
    for s in range(SLICES):
        _slice_pipe(s, x_ref, wpw_ref, wpb_ref, c_ref, thw_ref, thb_ref,
                    out_ref, d2_ref, h_ref)


def _slice_pipe(s, x_ref, wpw_ref, wpb_ref, c_ref, thw_ref, thb_ref, out_ref,
                d2_ref, h_ref):
    xb = x_ref[s]
    Z = _bdot(xb, wpw_ref[...], ((1,), (1,))) + wpb_ref[...]

    ones_col = jnp.ones((N, 1), dtype=jnp.float32)
    sq = jnp.sum(Z * Z, axis=1, keepdims=True)
    G = _bdot(Z, Z, ((1,), (1,)))
    sq_row = jnp.transpose(sq)
    d2_ref[s] = (sq - 2.0 * G) + sq_row

    INF = jnp.float32(jnp.inf)

    def _ce(u, v):
        return jnp.minimum(u, v), jnp.maximum(u, v)

    def _sorted4(t):
        q = N // 4
        a, b = _ce(t[:, :q], t[:, q:2 * q])
        c, d = _ce(t[:, 2 * q:3 * q], t[:, 3 * q:])
        a, c = _ce(a, c)
        b, d = _ce(b, d)
        b, c = _ce(b, c)
        w = q // 2
        while w >= 128:
            t1 = jnp.minimum(a[:, :w], d[:, w:])
            t2 = jnp.minimum(b[:, :w], c[:, w:])
            t3 = jnp.minimum(c[:, :w], b[:, w:])
            t4 = jnp.minimum(d[:, :w], a[:, w:])
            t1, t3 = _ce(t1, t3)
            t2, t4 = _ce(t2, t4)
            a, b = _ce(t1, t2)
            c, d = _ce(t3, t4)
            w //= 2
        return a, b, c, d

    def _extract(l1, l2, l3, l4, nstat):
        m = None
        for _ in range(nstat):
            m = jnp.min(l1, axis=1, keepdims=True)
            sh = l1 == m
            l1 = jnp.where(sh, l2, l1)
            l2 = jnp.where(sh, l3, l2)
            l3 = jnp.where(sh, l4, l3)
            l4 = jnp.where(sh, INF, l4)
        return m

    d2v = d2_ref[s]
    m = _extract(*_sorted4(d2v), 4)
    m = _extract(*_sorted4(jnp.where(d2v <= m, INF, d2v)), 4)
    m = _extract(*_sorted4(jnp.where(d2v <= m, INF, d2v)), 2)
    h_ref[s] = (d2v <= m).astype(jnp.bfloat16)

    S = _bdot(Z, c_ref[...], ((1,), (1,)))
    mx = jnp.max(S, axis=1, keepdims=True)
    e = jnp.exp(S - mx)
    Hc = e / jnp.sum(e, axis=1, keepdims=True)

    dv = jnp.float32(K) + jnp.sum(Hc, axis=1, keepdims=True)
    dvis = jax.lax.rsqrt(dv)

    Xt = _bdot(xb, thw_ref[...], ((1,), (1,))) + thb_ref[...]
    Xs = dvis * Xt

    h = h_ref[s]
    E_knn = _bdot(h, Xs, ((0,), (0,)))
    de_knn = _bdot(h, ones_col, ((0,), (0,)))
    E_knn = E_knn * jnp.where(de_knn > 0, 1.0 / jnp.where(de_knn > 0, de_knn, 1.0), 0.0)

    E_c = _bdot(Hc, Xs, ((0,), (0,)))
    de_c = _dot(Hc, ones_col, ((0,), (0,)))
    E_c = E_c * jnp.where(de_c > 0, 1.0 / jnp.where(de_c > 0, de_c, 1.0), 0.0)

    o = _bdot(h, E_knn, ((1,), (0,))) + _bdot(Hc, E_c, ((1,), (0,)))
    o = dvis * o
    out_ref[s] = jnp.where(o > 0, o, jnp.exp(jnp.minimum(o, 0.0)) - 1.0)


def kernel(x, Wp_w, Wp_b, C, Th_w, Th_b):
    Bb, Nn, Mm, Dd = x.shape
    BM = Bb * Mm
    x_flat = jnp.transpose(x, (0, 2, 1, 3)).reshape(BM, Nn, Dd)
    wpb = Wp_b.reshape(1, P)
    thb = Th_b.reshape(1, OUT)

    bcast = lambda *shape: pl.BlockSpec(shape, lambda i: (0,) * len(shape))
    y = pl.pallas_call(
        _tc_body,
        grid=(BM // SLICES,),
        in_specs=[
            pl.BlockSpec((SLICES, N, D), lambda i: (i, 0, 0)),
            bcast(P, D),
            bcast(1, P),
            bcast(NC, P),
            bcast(OUT, D),
            bcast(1, OUT),
        ],
        out_specs=pl.BlockSpec((SLICES, N, OUT), lambda i: (i, 0, 0)),
        out_shape=jax.ShapeDtypeStruct((BM, N, OUT), jnp.float32),
        scratch_shapes=[
            pltpu.VMEM((SLICES, N, N), jnp.float32),
            pltpu.VMEM((SLICES, N, N), jnp.bfloat16),
        ],
        compiler_params=pltpu.CompilerParams(
            dimension_semantics=("arbitrary",),
        ),
    )(x_flat, Wp_w, wpb, C, Th_w, thb)

    return jnp.transpose(y.reshape(Bb, Mm, Nn, OUT), (0, 2, 1, 3))
